# baseline (speedup 1.0000x reference)
import functools

import jax
import jax.numpy as jnp
from jax import lax
from jax.experimental import pallas as pl
from jax.experimental.pallas import tpu as pltpu
from jax.experimental.pallas import tpu_sc as plsc

RAYS = 65536
NSAMP = 64
NIMP = 128
NALL = NSAMP + NIMP
NWORK = 32
RPW = RAYS // NWORK
CH = 8
NCHUNK = RPW // CH

_f32 = jnp.float32
_i32 = jnp.int32


def _body(ro_h, rd_h, z_h, w_h, pts_h, za_h, zs_h,
          zin, win, oin, din, cf, bins, hist, hist2, samp, zall, ptsb):
    nc = 2
    wid = lax.axis_index("s") * nc + lax.axis_index("c")

    iota = lax.iota(_i32, 16)
    iota_f = iota.astype(_f32)
    ones_i = jnp.full((16,), 1, _i32)
    zero_i = jnp.full((16,), 0, _i32)

    def ray_body(r):
        rz = r * 64
        w0 = win[pl.ds(rz, 16)]
        w1 = win[pl.ds(rz + 16, 16)]
        w2 = win[pl.ds(rz + 32, 16)]
        w3 = win[pl.ds(rz + 48, 16)]
        eps = _f32(1e-5)
        a0 = jnp.where(iota >= 1, w0 + eps, _f32(0.0))
        a1 = w1 + eps
        a2 = w2 + eps
        a3 = jnp.where(iota <= 14, w3 + eps, _f32(0.0))
        c0 = plsc.cumsum(a0)
        c1 = plsc.cumsum(a1) + c0[15]
        c2 = plsc.cumsum(a2) + c1[15]
        c3 = plsc.cumsum(a3) + c2[15]
        total = c3[15]
        cf[pl.ds(0, 16)] = c0
        cf[pl.ds(16, 16)] = c1
        cf[pl.ds(32, 16)] = c2
        cf[pl.ds(48, 16)] = c3

        z0 = zin[pl.ds(rz, 16)]
        z1 = zin[pl.ds(rz + 16, 16)]
        z2 = zin[pl.ds(rz + 32, 16)]
        z3 = zin[pl.ds(rz + 48, 16)]
        zs0 = zin[pl.ds(rz + 1, 16)]
        zs1 = zin[pl.ds(rz + 17, 16)]
        zs2 = zin[pl.ds(rz + 33, 16)]
        zs3 = zin[pl.ds(rz + 49, 16)]
        half = _f32(0.5)
        bins[pl.ds(0, 16)] = half * (z0 + zs0)
        bins[pl.ds(16, 16)] = half * (z1 + zs1)
        bins[pl.ds(32, 16)] = half * (z2 + zs2)
        bins[pl.ds(48, 16)] = half * (z3 + zs3)

        for i in range(9):
            hist[pl.ds(16 * i, 16)] = zero_i
        tot_v = jnp.full((16,), 1.0, _f32) * total
        inv_v = _f32(1.0) / tot_v
        scale = _f32(127.0) * inv_v
        for i, cv in enumerate((c0, c1, c2, c3)):
            f = cv * scale
            ti = f.astype(_i32)
            m = ti + jnp.where(ti.astype(_f32) < f, 1, 0)
            m = jnp.minimum(m, 129)
            if i == 0:
                pass
            if i == 3:
                m = jnp.where(iota <= 14, m, 129)
            plsc.addupdate_scatter(hist, [m], ones_i)

        for i in range(5):
            hist2[pl.ds(16 * i, 16)] = zero_i
        td = total * _f32(1.0 / 127.0)
        eps_t = _f32(1e-5) * total
        rs = r * 128
        rza = r * 192
        carry = _i32(0)
        for i in range(8):
            h = hist[pl.ds(16 * i, 16)]
            inds = plsc.cumsum(h) + carry
            carry = inds[15]
            below = inds - 1
            above = jnp.minimum(inds, 62)
            cb = plsc.load_gather(cf, [below])
            ca = plsc.load_gather(cf, [above])
            bb = plsc.load_gather(bins, [below])
            ba = plsc.load_gather(bins, [above])
            u = (iota_f + _f32(16 * i)) * td
            denom = ca - cb
            rden = jnp.where(denom < eps_t, inv_v, _f32(1.0) / denom)
            t = (u - cb) * rden
            smp = bb + t * (ba - bb)
            samp[pl.ds(rs + 16 * i, 16)] = smp
            zb1 = plsc.load_gather(zin, [rz + inds])
            zb2 = plsc.load_gather(zin, [rz + jnp.minimum(inds + 1, 63)])
            r1 = inds + jnp.where(zb1 <= smp, 1, 0) + jnp.where(zb2 <= smp, 1, 0)
            q = iota + (16 * i) + r1
            plsc.store_scatter(zall, [rza + q], smp)
            plsc.addupdate_scatter(hist2, [r1], ones_i)

        carry = _i32(0)
        for i, zv in enumerate((z0, z1, z2, z3)):
            h2 = hist2[pl.ds(16 * i, 16)]
            sz = plsc.cumsum(h2) + carry
            carry = sz[15]
            p = iota + (16 * i) + sz
            plsc.store_scatter(zall, [rza + p], zv)

        r3 = r * 3
        ov = oin[pl.ds(r3, 16)]
        dv = din[pl.ds(r3, 16)]
        ox, oy, oz = ov[0], ov[1], ov[2]
        dx, dy, dz = dv[0], dv[1], dv[2]
        rp = r * 576
        for i in range(12):
            zv = zall[pl.ds(rza + 16 * i, 16)]
            mi = (iota + (16 * i)) * 3 + rp
            plsc.store_scatter(ptsb, [mi], ox + dx * zv)
            plsc.store_scatter(ptsb, [mi + 1], oy + dy * zv)
            plsc.store_scatter(ptsb, [mi + 2], oz + dz * zv)

    def chunk_body(ci, _):
        base = wid * RPW + ci * CH
        pltpu.sync_copy(z_h.at[pl.ds(base * 64, CH * 64)], zin.at[pl.ds(0, CH * 64)])
        pltpu.sync_copy(w_h.at[pl.ds(base * 64, CH * 64)], win)
        pltpu.sync_copy(ro_h.at[pl.ds(base * 3, CH * 3)], oin.at[pl.ds(0, CH * 3)])
        pltpu.sync_copy(rd_h.at[pl.ds(base * 3, CH * 3)], din.at[pl.ds(0, CH * 3)])
        for r in range(CH):
            ray_body(r)
        pltpu.sync_copy(samp, zs_h.at[pl.ds(base * 128, CH * 128)])
        pltpu.sync_copy(zall, za_h.at[pl.ds(base * 192, CH * 192)])
        pltpu.sync_copy(ptsb, pts_h.at[pl.ds(base * 576, CH * 576)])
        return _

    lax.fori_loop(0, NCHUNK, chunk_body, 0)


@functools.lru_cache(maxsize=1)
def _make_sc_call():
    mesh = plsc.VectorSubcoreMesh(
        core_axis_name="c", subcore_axis_name="s",
        num_cores=2, num_subcores=16)
    return pl.kernel(
        _body,
        out_type=[
            jax.ShapeDtypeStruct((RAYS * NALL * 3,), _f32),
            jax.ShapeDtypeStruct((RAYS * NALL,), _f32),
            jax.ShapeDtypeStruct((RAYS * NIMP,), _f32),
        ],
        mesh=mesh,
        compiler_params=pltpu.CompilerParams(needs_layout_passes=False),
        scratch_types=[
            pltpu.VMEM((CH * 64 + 16,), _f32),
            pltpu.VMEM((CH * 64,), _f32),
            pltpu.VMEM((CH * 3 + 16,), _f32),
            pltpu.VMEM((CH * 3 + 16,), _f32),
            pltpu.VMEM((64,), _f32),
            pltpu.VMEM((64,), _f32),
            pltpu.VMEM((144,), _i32),
            pltpu.VMEM((80,), _i32),
            pltpu.VMEM((CH * 128,), _f32),
            pltpu.VMEM((CH * 192,), _f32),
            pltpu.VMEM((CH * 576,), _f32),
        ],
    )


@jax.jit
def kernel(rays_o, rays_d, z_vals, weights):
    pts_f, za_f, zs_f = _make_sc_call()(
        rays_o.reshape(-1), rays_d.reshape(-1),
        z_vals.reshape(-1), weights.reshape(-1))
    return (pts_f.reshape(RAYS, NALL, 3),
            za_f.reshape(RAYS, NALL),
            zs_f.reshape(RAYS, NIMP))


# speedup vs baseline: 1.4981x; 1.4981x over previous
import functools

import jax
import jax.numpy as jnp
from jax import lax
from jax.experimental import pallas as pl
from jax.experimental.pallas import tpu as pltpu
from jax.experimental.pallas import tpu_sc as plsc

RAYS = 65536
NSAMP = 64
NIMP = 128
NALL = NSAMP + NIMP
NWORK = 32
RPW = RAYS // NWORK
CH = 8
NCHUNK = RPW // CH

_f32 = jnp.float32
_i32 = jnp.int32


def _body(z_h, w_h, za_h, zs_h,
          zin, win, cf, bins, hist, hist2, samp, zall):
    nc = 2
    wid = lax.axis_index("s") * nc + lax.axis_index("c")

    iota = lax.iota(_i32, 16)
    iota_f = iota.astype(_f32)
    ones_i = jnp.full((16,), 1, _i32)
    zero_i = jnp.full((16,), 0, _i32)

    def ray_body(r):
        rz = r * 64
        w0 = win[pl.ds(rz, 16)]
        w1 = win[pl.ds(rz + 16, 16)]
        w2 = win[pl.ds(rz + 32, 16)]
        w3 = win[pl.ds(rz + 48, 16)]
        eps = _f32(1e-5)
        a0 = jnp.where(iota >= 1, w0 + eps, _f32(0.0))
        a1 = w1 + eps
        a2 = w2 + eps
        a3 = jnp.where(iota <= 14, w3 + eps, _f32(0.0))
        c0 = plsc.cumsum(a0)
        c1 = plsc.cumsum(a1) + c0[15]
        c2 = plsc.cumsum(a2) + c1[15]
        c3 = plsc.cumsum(a3) + c2[15]
        total = c3[15]
        cf[pl.ds(0, 16)] = c0
        cf[pl.ds(16, 16)] = c1
        cf[pl.ds(32, 16)] = c2
        cf[pl.ds(48, 16)] = c3

        z0 = zin[pl.ds(rz, 16)]
        z1 = zin[pl.ds(rz + 16, 16)]
        z2 = zin[pl.ds(rz + 32, 16)]
        z3 = zin[pl.ds(rz + 48, 16)]
        zs0 = zin[pl.ds(rz + 1, 16)]
        zs1 = zin[pl.ds(rz + 17, 16)]
        zs2 = zin[pl.ds(rz + 33, 16)]
        zs3 = zin[pl.ds(rz + 49, 16)]
        half = _f32(0.5)
        bins[pl.ds(0, 16)] = half * (z0 + zs0)
        bins[pl.ds(16, 16)] = half * (z1 + zs1)
        bins[pl.ds(32, 16)] = half * (z2 + zs2)
        bins[pl.ds(48, 16)] = half * (z3 + zs3)

        for i in range(9):
            hist[pl.ds(16 * i, 16)] = zero_i
        tot_v = jnp.full((16,), 1.0, _f32) * total
        inv_v = _f32(1.0) / tot_v
        scale = _f32(127.0) * inv_v
        for i, cv in enumerate((c0, c1, c2, c3)):
            f = cv * scale
            ti = f.astype(_i32)
            m = ti + jnp.where(ti.astype(_f32) < f, 1, 0)
            m = jnp.minimum(m, 129)
            if i == 0:
                pass
            if i == 3:
                m = jnp.where(iota <= 14, m, 129)
            plsc.addupdate_scatter(hist, [m], ones_i)

        for i in range(5):
            hist2[pl.ds(16 * i, 16)] = zero_i
        td = total * _f32(1.0 / 127.0)
        eps_t = _f32(1e-5) * total
        rs = r * 128
        rza = r * 192
        carry = _i32(0)
        for i in range(8):
            h = hist[pl.ds(16 * i, 16)]
            inds = plsc.cumsum(h) + carry
            carry = inds[15]
            below = inds - 1
            above = jnp.minimum(inds, 62)
            cb = plsc.load_gather(cf, [below])
            ca = plsc.load_gather(cf, [above])
            bb = plsc.load_gather(bins, [below])
            ba = plsc.load_gather(bins, [above])
            u = (iota_f + _f32(16 * i)) * td
            denom = ca - cb
            rden = jnp.where(denom < eps_t, inv_v, _f32(1.0) / denom)
            t = (u - cb) * rden
            smp = bb + t * (ba - bb)
            samp[pl.ds(rs + 16 * i, 16)] = smp
            zb1 = plsc.load_gather(zin, [rz + inds])
            zb2 = plsc.load_gather(zin, [rz + jnp.minimum(inds + 1, 63)])
            r1 = inds + jnp.where(zb1 <= smp, 1, 0) + jnp.where(zb2 <= smp, 1, 0)
            q = iota + (16 * i) + r1
            plsc.store_scatter(zall, [rza + q], smp)
            plsc.addupdate_scatter(hist2, [r1], ones_i)

        carry = _i32(0)
        for i, zv in enumerate((z0, z1, z2, z3)):
            h2 = hist2[pl.ds(16 * i, 16)]
            sz = plsc.cumsum(h2) + carry
            carry = sz[15]
            p = iota + (16 * i) + sz
            plsc.store_scatter(zall, [rza + p], zv)

    def chunk_body(ci, _):
        base = wid * RPW + ci * CH
        pltpu.sync_copy(z_h.at[pl.ds(base * 64, CH * 64)], zin.at[pl.ds(0, CH * 64)])
        pltpu.sync_copy(w_h.at[pl.ds(base * 64, CH * 64)], win)
        for r in range(CH):
            ray_body(r)
        pltpu.sync_copy(samp, zs_h.at[pl.ds(base * 128, CH * 128)])
        pltpu.sync_copy(zall, za_h.at[pl.ds(base * 192, CH * 192)])
        return _

    lax.fori_loop(0, NCHUNK, chunk_body, 0)


@functools.lru_cache(maxsize=1)
def _make_sc_call():
    mesh = plsc.VectorSubcoreMesh(
        core_axis_name="c", subcore_axis_name="s",
        num_cores=2, num_subcores=16)
    return pl.kernel(
        _body,
        out_type=[
            jax.ShapeDtypeStruct((RAYS * NALL,), _f32),
            jax.ShapeDtypeStruct((RAYS * NIMP,), _f32),
        ],
        mesh=mesh,
        compiler_params=pltpu.CompilerParams(needs_layout_passes=False),
        scratch_types=[
            pltpu.VMEM((CH * 64 + 16,), _f32),
            pltpu.VMEM((CH * 64,), _f32),
            pltpu.VMEM((64,), _f32),
            pltpu.VMEM((64,), _f32),
            pltpu.VMEM((144,), _i32),
            pltpu.VMEM((80,), _i32),
            pltpu.VMEM((CH * 128,), _f32),
            pltpu.VMEM((CH * 192,), _f32),
        ],
    )


def _pts_body(o_ref, d_ref, z_ref, pts_ref):
    o = o_ref[...]
    d = d_ref[...]
    z = z_ref[...]
    pts_ref[...] = o[:, None, :] + d[:, None, :] * z[:, :, None]


_PB = 128


def _tc_pts(rays_o, rays_d, z2d):
    return pl.pallas_call(
        _pts_body,
        out_shape=jax.ShapeDtypeStruct((RAYS, NALL, 3), _f32),
        grid=(RAYS // _PB,),
        in_specs=[
            pl.BlockSpec((_PB, 3), lambda i: (i, 0)),
            pl.BlockSpec((_PB, 3), lambda i: (i, 0)),
            pl.BlockSpec((_PB, NALL), lambda i: (i, 0)),
        ],
        out_specs=pl.BlockSpec((_PB, NALL, 3), lambda i: (i, 0, 0)),
    )(rays_o, rays_d, z2d)


@jax.jit
def kernel(rays_o, rays_d, z_vals, weights):
    za_f, zs_f = _make_sc_call()(z_vals.reshape(-1), weights.reshape(-1))
    za = za_f.reshape(RAYS, NALL)
    pts = _tc_pts(rays_o, rays_d, za)
    return (pts, za, zs_f.reshape(RAYS, NIMP))


# speedup vs baseline: 5.4695x; 3.6509x over previous
import functools

import jax
import jax.numpy as jnp
from jax import lax
from jax.experimental import pallas as pl
from jax.experimental.pallas import tpu as pltpu
from jax.experimental.pallas import tpu_sc as plsc

RAYS = 65536
NSAMP = 64
NIMP = 128
NALL = NSAMP + NIMP
NWORK = 32
RPW = RAYS // NWORK
CH = 16
NCHUNK = RPW // CH

_f32 = jnp.float32
_i32 = jnp.int32


def _body(z_h, w_h, za_h, zs_h,
          zin, win, cf, bins, hist, hist2, samp, zall):
    nc = 2
    wid = lax.axis_index("s") * nc + lax.axis_index("c")

    iota = lax.iota(_i32, 16)
    iota_f = iota.astype(_f32)
    ones_i = jnp.full((16,), 1, _i32)
    zero_i = jnp.full((16,), 0, _i32)

    def ray_body(r):
        rz = r * 64
        w0 = win[pl.ds(rz, 16)]
        w1 = win[pl.ds(rz + 16, 16)]
        w2 = win[pl.ds(rz + 32, 16)]
        w3 = win[pl.ds(rz + 48, 16)]
        eps = _f32(1e-5)
        a0 = jnp.where(iota >= 1, w0 + eps, _f32(0.0))
        a1 = w1 + eps
        a2 = w2 + eps
        a3 = jnp.where(iota <= 14, w3 + eps, _f32(0.0))
        c0 = plsc.cumsum(a0)
        c1 = plsc.cumsum(a1) + c0[15]
        c2 = plsc.cumsum(a2) + c1[15]
        c3 = plsc.cumsum(a3) + c2[15]
        total = c3[15]
        cf[pl.ds(0, 16)] = c0
        cf[pl.ds(16, 16)] = c1
        cf[pl.ds(32, 16)] = c2
        cf[pl.ds(48, 16)] = c3

        z0 = zin[pl.ds(rz, 16)]
        z1 = zin[pl.ds(rz + 16, 16)]
        z2 = zin[pl.ds(rz + 32, 16)]
        z3 = zin[pl.ds(rz + 48, 16)]
        zs0 = zin[pl.ds(rz + 1, 16)]
        zs1 = zin[pl.ds(rz + 17, 16)]
        zs2 = zin[pl.ds(rz + 33, 16)]
        zs3 = zin[pl.ds(rz + 49, 16)]
        half = _f32(0.5)
        bins[pl.ds(0, 16)] = half * (z0 + zs0)
        bins[pl.ds(16, 16)] = half * (z1 + zs1)
        bins[pl.ds(32, 16)] = half * (z2 + zs2)
        bins[pl.ds(48, 16)] = half * (z3 + zs3)

        for i in range(9):
            hist[pl.ds(16 * i, 16)] = zero_i
        tot_v = jnp.full((16,), 1.0, _f32) * total
        inv_v = _f32(1.0) / tot_v
        scale = _f32(127.0) * inv_v
        for i, cv in enumerate((c0, c1, c2, c3)):
            f = cv * scale
            ti = f.astype(_i32)
            m = ti + jnp.where(ti.astype(_f32) < f, 1, 0)
            m = jnp.minimum(m, 129)
            if i == 0:
                pass
            if i == 3:
                m = jnp.where(iota <= 14, m, 129)
            plsc.addupdate_scatter(hist, [m], ones_i)

        for i in range(5):
            hist2[pl.ds(16 * i, 16)] = zero_i
        td = total * _f32(1.0 / 127.0)
        eps_t = _f32(1e-5) * total
        rfull = jnp.full((16,), r, _i32)
        carry = _i32(0)
        for i in range(8):
            h = hist[pl.ds(16 * i, 16)]
            inds = plsc.cumsum(h) + carry
            carry = inds[15]
            below = inds - 1
            above = jnp.minimum(inds, 62)
            cb = plsc.load_gather(cf, [below])
            ca = plsc.load_gather(cf, [above])
            bb = plsc.load_gather(bins, [below])
            ba = plsc.load_gather(bins, [above])
            u = (iota_f + _f32(16 * i)) * td
            denom = ca - cb
            rden = jnp.where(denom < eps_t, inv_v, _f32(1.0) / denom)
            t = (u - cb) * rden
            smp = bb + t * (ba - bb)
            plsc.store_scatter(samp, [iota + (16 * i), rfull], smp)
            zb1 = plsc.load_gather(zin, [rz + inds])
            zb2 = plsc.load_gather(zin, [rz + jnp.minimum(inds + 1, 63)])
            r1 = inds + jnp.where(zb1 <= smp, 1, 0) + jnp.where(zb2 <= smp, 1, 0)
            q = iota + (16 * i) + r1
            plsc.store_scatter(zall, [q, rfull], smp)
            plsc.addupdate_scatter(hist2, [r1], ones_i)

        carry = _i32(0)
        for i, zv in enumerate((z0, z1, z2, z3)):
            h2 = hist2[pl.ds(16 * i, 16)]
            sz = plsc.cumsum(h2) + carry
            carry = sz[15]
            p = iota + (16 * i) + sz
            plsc.store_scatter(zall, [p, rfull], zv)

    def chunk_body(ci, _):
        base = wid * RPW + ci * CH
        pltpu.sync_copy(z_h.at[pl.ds(base * 64, CH * 64)], zin.at[pl.ds(0, CH * 64)])
        pltpu.sync_copy(w_h.at[pl.ds(base * 64, CH * 64)], win)
        for r in range(CH):
            ray_body(r)
        pltpu.sync_copy(samp, zs_h.at[:, pl.ds(base, CH)])
        pltpu.sync_copy(zall, za_h.at[:, pl.ds(base, CH)])
        return _

    lax.fori_loop(0, NCHUNK, chunk_body, 0)


@functools.lru_cache(maxsize=1)
def _make_sc_call():
    mesh = plsc.VectorSubcoreMesh(
        core_axis_name="c", subcore_axis_name="s",
        num_cores=2, num_subcores=16)
    return pl.kernel(
        _body,
        out_type=[
            jax.ShapeDtypeStruct((NALL, RAYS), _f32),
            jax.ShapeDtypeStruct((NIMP, RAYS), _f32),
        ],
        mesh=mesh,
        compiler_params=pltpu.CompilerParams(
            needs_layout_passes=False, use_tc_tiling_on_sc=False),
        scratch_types=[
            pltpu.VMEM((CH * 64 + 16,), _f32),
            pltpu.VMEM((CH * 64,), _f32),
            pltpu.VMEM((64,), _f32),
            pltpu.VMEM((64,), _f32),
            pltpu.VMEM((144,), _i32),
            pltpu.VMEM((80,), _i32),
            pltpu.VMEM((NIMP, CH), _f32),
            pltpu.VMEM((NALL, CH), _f32),
        ],
    )


def _pts_body(o_ref, d_ref, z_ref, pts_ref):
    z = z_ref[...]
    for c in range(3):
        o = o_ref[c, :][None, :]
        d = d_ref[c, :][None, :]
        pts_ref[c, :, :] = o + d * z


_PB = 512


def _tc_pts(ro_t, rd_t, za_t):
    return pl.pallas_call(
        _pts_body,
        out_shape=jax.ShapeDtypeStruct((3, NALL, RAYS), _f32),
        grid=(RAYS // _PB,),
        in_specs=[
            pl.BlockSpec((3, _PB), lambda i: (0, i)),
            pl.BlockSpec((3, _PB), lambda i: (0, i)),
            pl.BlockSpec((NALL, _PB), lambda i: (0, i)),
        ],
        out_specs=pl.BlockSpec((3, NALL, _PB), lambda i: (0, 0, i)),
    )(ro_t, rd_t, za_t)


@jax.jit
def kernel(rays_o, rays_d, z_vals, weights):
    za_t, zs_t = _make_sc_call()(z_vals.reshape(-1), weights.reshape(-1))
    pts_t = _tc_pts(rays_o.T, rays_d.T, za_t)
    return (jnp.transpose(pts_t, (2, 1, 0)), za_t.T, zs_t.T)


# speedup vs baseline: 6.7307x; 1.2306x over previous
import functools

import jax
import jax.numpy as jnp
from jax import lax
from jax.experimental import pallas as pl
from jax.experimental.pallas import tpu as pltpu
from jax.experimental.pallas import tpu_sc as plsc

RAYS = 65536
NSAMP = 64
NIMP = 128
NALL = NSAMP + NIMP
NWORK = 32
RPW = RAYS // NWORK
CH = 8
NCHUNK = RPW // CH

_f32 = jnp.float32
_i32 = jnp.int32


def _body(z_h, w_h, za_h, zs_h,
          zin0, win0, samp0, zall0,
          zin1, win1, samp1, zall1,
          cf, bins, hist, hist2,
          sem_in0, sem_out0, sem_in1, sem_out1):
    nc = 2
    wid = lax.axis_index("s") * nc + lax.axis_index("c")

    iota = lax.iota(_i32, 16)
    iota_f = iota.astype(_f32)
    ones_i = jnp.full((16,), 1, _i32)
    zero_i = jnp.full((16,), 0, _i32)

    def ray_body(r, zin, win, samp, zall):
        rz = r * 64
        w0 = win[pl.ds(rz, 16)]
        w1 = win[pl.ds(rz + 16, 16)]
        w2 = win[pl.ds(rz + 32, 16)]
        w3 = win[pl.ds(rz + 48, 16)]
        eps = _f32(1e-5)
        a0 = jnp.where(iota >= 1, w0 + eps, _f32(0.0))
        a1 = w1 + eps
        a2 = w2 + eps
        a3 = jnp.where(iota <= 14, w3 + eps, _f32(0.0))
        c0 = plsc.cumsum(a0)
        c1 = plsc.cumsum(a1) + c0[15]
        c2 = plsc.cumsum(a2) + c1[15]
        c3 = plsc.cumsum(a3) + c2[15]
        total = c3[15]
        cf[pl.ds(0, 16)] = c0
        cf[pl.ds(16, 16)] = c1
        cf[pl.ds(32, 16)] = c2
        cf[pl.ds(48, 16)] = c3

        z0 = zin[pl.ds(rz, 16)]
        z1 = zin[pl.ds(rz + 16, 16)]
        z2 = zin[pl.ds(rz + 32, 16)]
        z3 = zin[pl.ds(rz + 48, 16)]
        zs0 = zin[pl.ds(rz + 1, 16)]
        zs1 = zin[pl.ds(rz + 17, 16)]
        zs2 = zin[pl.ds(rz + 33, 16)]
        zs3 = zin[pl.ds(rz + 49, 16)]
        half = _f32(0.5)
        bins[pl.ds(0, 16)] = half * (z0 + zs0)
        bins[pl.ds(16, 16)] = half * (z1 + zs1)
        bins[pl.ds(32, 16)] = half * (z2 + zs2)
        bins[pl.ds(48, 16)] = half * (z3 + zs3)

        for i in range(9):
            hist[pl.ds(16 * i, 16)] = zero_i
        tot_v = jnp.full((16,), 1.0, _f32) * total
        inv_v = _f32(1.0) / tot_v
        scale = _f32(127.0) * inv_v
        for i, cv in enumerate((c0, c1, c2, c3)):
            f = cv * scale
            ti = f.astype(_i32)
            m = ti + jnp.where(ti.astype(_f32) < f, 1, 0)
            m = jnp.minimum(m, 129)
            if i == 0:
                pass
            if i == 3:
                m = jnp.where(iota <= 14, m, 129)
            plsc.addupdate_scatter(hist, [m], ones_i)

        for i in range(5):
            hist2[pl.ds(16 * i, 16)] = zero_i
        td = total * _f32(1.0 / 127.0)
        eps_t = _f32(1e-5) * total
        rfull = jnp.full((16,), r, _i32)
        carry = _i32(0)
        for i in range(8):
            h = hist[pl.ds(16 * i, 16)]
            inds = plsc.cumsum(h) + carry
            carry = inds[15]
            below = inds - 1
            above = jnp.minimum(inds, 62)
            cb = plsc.load_gather(cf, [below])
            ca = plsc.load_gather(cf, [above])
            bb = plsc.load_gather(bins, [below])
            ba = plsc.load_gather(bins, [above])
            u = (iota_f + _f32(16 * i)) * td
            denom = ca - cb
            rden = jnp.where(denom < eps_t, inv_v, _f32(1.0) / denom)
            t = (u - cb) * rden
            smp = bb + t * (ba - bb)
            plsc.store_scatter(samp, [iota + (16 * i), rfull], smp)
            zb1 = plsc.load_gather(zin, [rz + inds])
            zb2 = plsc.load_gather(zin, [rz + jnp.minimum(inds + 1, 63)])
            r1 = inds + jnp.where(zb1 <= smp, 1, 0) + jnp.where(zb2 <= smp, 1, 0)
            q = iota + (16 * i) + r1
            plsc.store_scatter(zall, [q, rfull], smp)
            plsc.addupdate_scatter(hist2, [r1], ones_i)

        carry = _i32(0)
        for i, zv in enumerate((z0, z1, z2, z3)):
            h2 = hist2[pl.ds(16 * i, 16)]
            sz = plsc.cumsum(h2) + carry
            carry = sz[15]
            p = iota + (16 * i) + sz
            plsc.store_scatter(zall, [p, rfull], zv)

    bufs = (
        (zin0, win0, samp0, zall0, sem_in0, sem_out0),
        (zin1, win1, samp1, zall1, sem_in1, sem_out1),
    )

    def issue_in(ci, zin, win, sem_in):
        base = wid * RPW + ci * CH
        pltpu.async_copy(z_h.at[pl.ds(base * 64, CH * 64)],
                         zin.at[pl.ds(0, CH * 64)], sem_in)
        pltpu.async_copy(w_h.at[pl.ds(base * 64, CH * 64)], win, sem_in)

    def wait_in(zin, win, sem_in):
        pltpu.make_async_copy(z_h.at[pl.ds(0, CH * 64)],
                              zin.at[pl.ds(0, CH * 64)], sem_in).wait()
        pltpu.make_async_copy(w_h.at[pl.ds(0, CH * 64)], win, sem_in).wait()

    def issue_out(ci, samp, zall, sem_out):
        base = wid * RPW + ci * CH
        pltpu.async_copy(samp, zs_h.at[:, pl.ds(base, CH)], sem_out)
        pltpu.async_copy(zall, za_h.at[:, pl.ds(base, CH)], sem_out)

    def wait_out(samp, zall, sem_out):
        pltpu.make_async_copy(samp, zs_h.at[:, pl.ds(0, CH)], sem_out).wait()
        pltpu.make_async_copy(zall, za_h.at[:, pl.ds(0, CH)], sem_out).wait()

    issue_in(0, zin0, win0, sem_in0)

    def pair_body(j, _):
        for b in (0, 1):
            ci = 2 * j + b
            zin, win, samp, zall, sem_in, sem_out = bufs[b]
            nzin, nwin, _ns, _nz, nsem_in, _nso = bufs[1 - b]
            wait_in(zin, win, sem_in)

            @pl.when(ci + 1 < NCHUNK)
            def _prefetch():
                issue_in(ci + 1, nzin, nwin, nsem_in)

            @pl.when(ci >= 2)
            def _drain():
                wait_out(samp, zall, sem_out)

            for r in range(CH):
                ray_body(r, zin, win, samp, zall)
            issue_out(ci, samp, zall, sem_out)
        return _

    lax.fori_loop(0, NCHUNK // 2, pair_body, 0)
    wait_out(samp0, zall0, sem_out0)
    wait_out(samp1, zall1, sem_out1)


@functools.lru_cache(maxsize=1)
def _make_sc_call():
    mesh = plsc.VectorSubcoreMesh(
        core_axis_name="c", subcore_axis_name="s",
        num_cores=2, num_subcores=16)
    return pl.kernel(
        _body,
        out_type=[
            jax.ShapeDtypeStruct((NALL, RAYS), _f32),
            jax.ShapeDtypeStruct((NIMP, RAYS), _f32),
        ],
        mesh=mesh,
        compiler_params=pltpu.CompilerParams(
            needs_layout_passes=False, use_tc_tiling_on_sc=False),
        scratch_types=[
            pltpu.VMEM((CH * 64 + 16,), _f32),
            pltpu.VMEM((CH * 64,), _f32),
            pltpu.VMEM((NIMP, CH), _f32),
            pltpu.VMEM((NALL, CH), _f32),
            pltpu.VMEM((CH * 64 + 16,), _f32),
            pltpu.VMEM((CH * 64,), _f32),
            pltpu.VMEM((NIMP, CH), _f32),
            pltpu.VMEM((NALL, CH), _f32),
            pltpu.VMEM((64,), _f32),
            pltpu.VMEM((64,), _f32),
            pltpu.VMEM((144,), _i32),
            pltpu.VMEM((80,), _i32),
            pltpu.SemaphoreType.DMA,
            pltpu.SemaphoreType.DMA,
            pltpu.SemaphoreType.DMA,
            pltpu.SemaphoreType.DMA,
        ],
    )


def _pts_body(o_ref, d_ref, z_ref, pts_ref):
    z = z_ref[...]
    for c in range(3):
        o = o_ref[c, :][None, :]
        d = d_ref[c, :][None, :]
        pts_ref[c, :, :] = o + d * z


_PB = 512


def _tc_pts(ro_t, rd_t, za_t):
    return pl.pallas_call(
        _pts_body,
        out_shape=jax.ShapeDtypeStruct((3, NALL, RAYS), _f32),
        grid=(RAYS // _PB,),
        in_specs=[
            pl.BlockSpec((3, _PB), lambda i: (0, i)),
            pl.BlockSpec((3, _PB), lambda i: (0, i)),
            pl.BlockSpec((NALL, _PB), lambda i: (0, i)),
        ],
        out_specs=pl.BlockSpec((3, NALL, _PB), lambda i: (0, 0, i)),
    )(ro_t, rd_t, za_t)


@jax.jit
def kernel(rays_o, rays_d, z_vals, weights):
    za_t, zs_t = _make_sc_call()(z_vals.reshape(-1), weights.reshape(-1))
    pts_t = _tc_pts(rays_o.T, rays_d.T, za_t)
    return (jnp.transpose(pts_t, (2, 1, 0)), za_t.T, zs_t.T)


# speedup vs baseline: 6.7624x; 1.0047x over previous
import functools

import jax
import jax.numpy as jnp
from jax import lax
from jax.experimental import pallas as pl
from jax.experimental.pallas import tpu as pltpu
from jax.experimental.pallas import tpu_sc as plsc

RAYS = 65536
NSAMP = 64
NIMP = 128
NALL = NSAMP + NIMP
NWORK = 32
RPW = RAYS // NWORK
CH = 8
NCHUNK = RPW // CH

_f32 = jnp.float32
_i32 = jnp.int32


def _body(z_h, w_h, za_h, zs_h,
          zin0, win0, samp0, zall0,
          zin1, win1, samp1, zall1,
          cf, bins, hist, hist2,
          sem_in0, sem_out0, sem_in1, sem_out1):
    nc = 2
    wid = lax.axis_index("s") * nc + lax.axis_index("c")

    iota = lax.iota(_i32, 16)
    iota_f = iota.astype(_f32)
    ones_i = jnp.full((16,), 1, _i32)
    zero_i = jnp.full((16,), 0, _i32)

    def ray_body(r, zin, win, samp, zall):
        rz = r * 64
        w0 = win[pl.ds(rz, 16)]
        w1 = win[pl.ds(rz + 16, 16)]
        w2 = win[pl.ds(rz + 32, 16)]
        w3 = win[pl.ds(rz + 48, 16)]
        eps = _f32(1e-5)
        a0 = jnp.where(iota >= 1, w0 + eps, _f32(0.0))
        a1 = w1 + eps
        a2 = w2 + eps
        a3 = jnp.where(iota <= 14, w3 + eps, _f32(0.0))
        c0 = plsc.cumsum(a0)
        c1 = plsc.cumsum(a1) + c0[15]
        c2 = plsc.cumsum(a2) + c1[15]
        c3 = plsc.cumsum(a3) + c2[15]
        total = c3[15]
        cf[pl.ds(0, 16)] = c0
        cf[pl.ds(16, 16)] = c1
        cf[pl.ds(32, 16)] = c2
        cf[pl.ds(48, 16)] = c3

        z0 = zin[pl.ds(rz, 16)]
        z1 = zin[pl.ds(rz + 16, 16)]
        z2 = zin[pl.ds(rz + 32, 16)]
        z3 = zin[pl.ds(rz + 48, 16)]
        zs0 = zin[pl.ds(rz + 1, 16)]
        zs1 = zin[pl.ds(rz + 17, 16)]
        zs2 = zin[pl.ds(rz + 33, 16)]
        zs3 = zin[pl.ds(rz + 49, 16)]
        half = _f32(0.5)
        bins[pl.ds(0, 16)] = half * (z0 + zs0)
        bins[pl.ds(16, 16)] = half * (z1 + zs1)
        bins[pl.ds(32, 16)] = half * (z2 + zs2)
        bins[pl.ds(48, 16)] = half * (z3 + zs3)

        for i in range(8):
            hist[pl.ds(16 * i, 16)] = zero_i
        tot_v = jnp.full((16,), 1.0, _f32) * total
        inv_v = _f32(1.0) / tot_v
        scale = _f32(127.0) * inv_v
        for i, cv in enumerate((c0, c1, c2, c3)):
            f = cv * scale
            m = f.astype(_i32) + 1
            if i == 0:
                m = jnp.where(iota >= 1, m, 0)
            if i == 3:
                m = jnp.where(iota <= 14, m, 129)
            plsc.addupdate_scatter(hist, [m], ones_i)

        for i in range(4):
            hist2[pl.ds(16 * i, 16)] = zero_i
        td = total * _f32(1.0 / 127.0)
        eps_t = _f32(1e-5) * total
        rfull = jnp.full((16,), r, _i32)
        carry = _i32(0)
        for i in range(8):
            h = hist[pl.ds(16 * i, 16)]
            inds = plsc.cumsum(h) + carry
            carry = inds[15]
            below = inds - 1
            above = jnp.minimum(inds, 62)
            cb = plsc.load_gather(cf, [below])
            ca = plsc.load_gather(cf, [above])
            bb = plsc.load_gather(bins, [below])
            ba = plsc.load_gather(bins, [above])
            u = (iota_f + _f32(16 * i)) * td
            denom = ca - cb
            rden = jnp.where(denom < eps_t, inv_v, _f32(1.0) / denom)
            t = (u - cb) * rden
            smp = bb + t * (ba - bb)
            plsc.store_scatter(samp, [iota + (16 * i), rfull], smp)
            zb1 = plsc.load_gather(zin, [rz + inds])
            zb2 = plsc.load_gather(zin, [rz + jnp.minimum(inds + 1, 63)])
            r1 = inds + jnp.where(zb1 <= smp, 1, 0) + jnp.where(zb2 <= smp, 1, 0)
            q = iota + (16 * i) + r1
            plsc.store_scatter(zall, [q, rfull], smp)
            plsc.addupdate_scatter(hist2, [r1], ones_i)

        carry = _i32(0)
        for i, zv in enumerate((z0, z1, z2, z3)):
            h2 = hist2[pl.ds(16 * i, 16)]
            sz = plsc.cumsum(h2) + carry
            carry = sz[15]
            p = iota + (16 * i) + sz
            plsc.store_scatter(zall, [p, rfull], zv)

    bufs = (
        (zin0, win0, samp0, zall0, sem_in0, sem_out0),
        (zin1, win1, samp1, zall1, sem_in1, sem_out1),
    )

    def issue_in(ci, zin, win, sem_in):
        base = wid * RPW + ci * CH
        pltpu.async_copy(z_h.at[pl.ds(base * 64, CH * 64)],
                         zin.at[pl.ds(0, CH * 64)], sem_in)
        pltpu.async_copy(w_h.at[pl.ds(base * 64, CH * 64)], win, sem_in)

    def wait_in(zin, win, sem_in):
        pltpu.make_async_copy(z_h.at[pl.ds(0, CH * 64)],
                              zin.at[pl.ds(0, CH * 64)], sem_in).wait()
        pltpu.make_async_copy(w_h.at[pl.ds(0, CH * 64)], win, sem_in).wait()

    def issue_out(ci, samp, zall, sem_out):
        base = wid * RPW + ci * CH
        pltpu.async_copy(samp, zs_h.at[:, pl.ds(base, CH)], sem_out)
        pltpu.async_copy(zall, za_h.at[:, pl.ds(base, CH)], sem_out)

    def wait_out(samp, zall, sem_out):
        pltpu.make_async_copy(samp, zs_h.at[:, pl.ds(0, CH)], sem_out).wait()
        pltpu.make_async_copy(zall, za_h.at[:, pl.ds(0, CH)], sem_out).wait()

    issue_in(0, zin0, win0, sem_in0)

    def pair_body(j, _):
        for b in (0, 1):
            ci = 2 * j + b
            zin, win, samp, zall, sem_in, sem_out = bufs[b]
            nzin, nwin, _ns, _nz, nsem_in, _nso = bufs[1 - b]
            wait_in(zin, win, sem_in)

            @pl.when(ci + 1 < NCHUNK)
            def _prefetch():
                issue_in(ci + 1, nzin, nwin, nsem_in)

            @pl.when(ci >= 2)
            def _drain():
                wait_out(samp, zall, sem_out)

            for r in range(CH):
                ray_body(r, zin, win, samp, zall)
            issue_out(ci, samp, zall, sem_out)
        return _

    lax.fori_loop(0, NCHUNK // 2, pair_body, 0)
    wait_out(samp0, zall0, sem_out0)
    wait_out(samp1, zall1, sem_out1)


@functools.lru_cache(maxsize=1)
def _make_sc_call():
    mesh = plsc.VectorSubcoreMesh(
        core_axis_name="c", subcore_axis_name="s",
        num_cores=2, num_subcores=16)
    return pl.kernel(
        _body,
        out_type=[
            jax.ShapeDtypeStruct((NALL, RAYS), _f32),
            jax.ShapeDtypeStruct((NIMP, RAYS), _f32),
        ],
        mesh=mesh,
        compiler_params=pltpu.CompilerParams(
            needs_layout_passes=False, use_tc_tiling_on_sc=False),
        scratch_types=[
            pltpu.VMEM((CH * 64 + 16,), _f32),
            pltpu.VMEM((CH * 64,), _f32),
            pltpu.VMEM((NIMP, CH), _f32),
            pltpu.VMEM((NALL, CH), _f32),
            pltpu.VMEM((CH * 64 + 16,), _f32),
            pltpu.VMEM((CH * 64,), _f32),
            pltpu.VMEM((NIMP, CH), _f32),
            pltpu.VMEM((NALL, CH), _f32),
            pltpu.VMEM((64,), _f32),
            pltpu.VMEM((64,), _f32),
            pltpu.VMEM((144,), _i32),
            pltpu.VMEM((80,), _i32),
            pltpu.SemaphoreType.DMA,
            pltpu.SemaphoreType.DMA,
            pltpu.SemaphoreType.DMA,
            pltpu.SemaphoreType.DMA,
        ],
    )


def _pts_body(o_ref, d_ref, z_ref, pts_ref):
    z = z_ref[...]
    for c in range(3):
        o = o_ref[c, :][None, :]
        d = d_ref[c, :][None, :]
        pts_ref[c, :, :] = o + d * z


_PB = 512


def _tc_pts(ro_t, rd_t, za_t):
    return pl.pallas_call(
        _pts_body,
        out_shape=jax.ShapeDtypeStruct((3, NALL, RAYS), _f32),
        grid=(RAYS // _PB,),
        in_specs=[
            pl.BlockSpec((3, _PB), lambda i: (0, i)),
            pl.BlockSpec((3, _PB), lambda i: (0, i)),
            pl.BlockSpec((NALL, _PB), lambda i: (0, i)),
        ],
        out_specs=pl.BlockSpec((3, NALL, _PB), lambda i: (0, 0, i)),
    )(ro_t, rd_t, za_t)


@jax.jit
def kernel(rays_o, rays_d, z_vals, weights):
    za_t, zs_t = _make_sc_call()(z_vals.reshape(-1), weights.reshape(-1))
    pts_t = _tc_pts(rays_o.T, rays_d.T, za_t)
    return (jnp.transpose(pts_t, (2, 1, 0)), za_t.T, zs_t.T)


# speedup vs baseline: 6.9765x; 1.0317x over previous
import functools

import jax
import jax.numpy as jnp
from jax import lax
from jax.experimental import pallas as pl
from jax.experimental.pallas import tpu as pltpu
from jax.experimental.pallas import tpu_sc as plsc

RAYS = 65536
NSAMP = 64
NIMP = 128
NALL = NSAMP + NIMP
NWORK = 32
RPW = RAYS // NWORK
CH = 8
NCHUNK = RPW // CH

_f32 = jnp.float32
_i32 = jnp.int32


def _body(z_h, w_h, za_h, zs_h,
          zin0, win0, samp0, zall0,
          zin1, win1, samp1, zall1,
          cf, bins, hist, hist2,
          sem_in0, sem_out0, sem_in1, sem_out1):
    nc = 2
    wid = lax.axis_index("s") * nc + lax.axis_index("c")

    iota = lax.iota(_i32, 16)
    iota_f = iota.astype(_f32)
    ones_i = jnp.full((16,), 1, _i32)
    zero_i = jnp.full((16,), 0, _i32)

    def ray_body(r, zin, win, samp, zall):
        rz = r * 64
        w0 = win[pl.ds(rz, 16)]
        w1 = win[pl.ds(rz + 16, 16)]
        w2 = win[pl.ds(rz + 32, 16)]
        w3 = win[pl.ds(rz + 48, 16)]
        eps = _f32(1e-5)
        a0 = jnp.where(iota >= 1, w0 + eps, _f32(0.0))
        a1 = w1 + eps
        a2 = w2 + eps
        a3 = jnp.where(iota <= 14, w3 + eps, _f32(0.0))
        c0 = plsc.cumsum(a0)
        c1 = plsc.cumsum(a1) + c0[15]
        c2 = plsc.cumsum(a2) + c1[15]
        c3 = plsc.cumsum(a3) + c2[15]
        total = c3[15]
        cf[pl.ds(0, 16)] = c0
        cf[pl.ds(16, 16)] = c1
        cf[pl.ds(32, 16)] = c2
        cf[pl.ds(48, 16)] = c3

        z0 = zin[pl.ds(rz, 16)]
        z1 = zin[pl.ds(rz + 16, 16)]
        z2 = zin[pl.ds(rz + 32, 16)]
        z3 = zin[pl.ds(rz + 48, 16)]
        zs0 = zin[pl.ds(rz + 1, 16)]
        zs1 = zin[pl.ds(rz + 17, 16)]
        zs2 = zin[pl.ds(rz + 33, 16)]
        zs3 = zin[pl.ds(rz + 49, 16)]
        half = _f32(0.5)
        bins[pl.ds(0, 16)] = half * (z0 + zs0)
        bins[pl.ds(16, 16)] = half * (z1 + zs1)
        bins[pl.ds(32, 16)] = half * (z2 + zs2)
        bins[pl.ds(48, 16)] = half * (z3 + zs3)

        for i in range(8):
            hist[pl.ds(16 * i, 16)] = zero_i
        tot_v = jnp.full((16,), 1.0, _f32) * total
        inv_v = _f32(1.0) / tot_v
        scale = _f32(127.0) * inv_v
        for i, cv in enumerate((c0, c1, c2, c3)):
            f = cv * scale
            m = f.astype(_i32) + 1
            if i == 0:
                m = jnp.where(iota >= 1, m, 0)
            if i == 3:
                m = jnp.where(iota <= 14, m, 129)
            plsc.addupdate_scatter(hist, [m], ones_i)

        for i in range(4):
            hist2[pl.ds(16 * i, 16)] = zero_i
        td = total * _f32(1.0 / 127.0)
        eps_t = _f32(1e-5) * total
        rfull = jnp.full((16,), r, _i32)
        carry = _i32(0)
        for i in range(8):
            h = hist[pl.ds(16 * i, 16)]
            inds = plsc.cumsum(h) + carry
            carry = inds[15]
            below = inds - 1
            above = jnp.minimum(inds, 62)
            cb = plsc.load_gather(cf, [below])
            ca = plsc.load_gather(cf, [above])
            bb = plsc.load_gather(bins, [below])
            ba = plsc.load_gather(bins, [above])
            u = (iota_f + _f32(16 * i)) * td
            denom = ca - cb
            rden = jnp.where(denom < eps_t, inv_v, _f32(1.0) / denom)
            t = (u - cb) * rden
            smp = bb + t * (ba - bb)
            plsc.store_scatter(samp, [iota + (16 * i), rfull], smp)
            zb1 = plsc.load_gather(zin, [rz + inds])
            zb2 = plsc.load_gather(zin, [rz + jnp.minimum(inds + 1, 63)])
            r1 = inds + jnp.where(zb1 <= smp, 1, 0) + jnp.where(zb2 <= smp, 1, 0)
            q = iota + (16 * i) + r1
            plsc.store_scatter(zall, [q, rfull], smp)
            plsc.addupdate_scatter(hist2, [r1], ones_i)

        carry = _i32(0)
        for i, zv in enumerate((z0, z1, z2, z3)):
            h2 = hist2[pl.ds(16 * i, 16)]
            sz = plsc.cumsum(h2) + carry
            carry = sz[15]
            p = iota + (16 * i) + sz
            plsc.store_scatter(zall, [p, rfull], zv)

    bufs = (
        (zin0, win0, samp0, zall0, sem_in0, sem_out0),
        (zin1, win1, samp1, zall1, sem_in1, sem_out1),
    )

    def issue_in(ci, zin, win, sem_in):
        base = wid * RPW + ci * CH
        pltpu.async_copy(z_h.at[pl.ds(base * 64, CH * 64)],
                         zin.at[pl.ds(0, CH * 64)], sem_in)
        pltpu.async_copy(w_h.at[pl.ds(base * 64, CH * 64)], win, sem_in)

    def wait_in(zin, win, sem_in):
        pltpu.make_async_copy(z_h.at[pl.ds(0, CH * 64)],
                              zin.at[pl.ds(0, CH * 64)], sem_in).wait()
        pltpu.make_async_copy(w_h.at[pl.ds(0, CH * 64)], win, sem_in).wait()

    def issue_out(ci, samp, zall, sem_out):
        base = wid * RPW + ci * CH
        pltpu.async_copy(samp, zs_h.at[:, pl.ds(base, CH)], sem_out)
        pltpu.async_copy(zall, za_h.at[:, pl.ds(base, CH)], sem_out)

    def wait_out(samp, zall, sem_out):
        pltpu.make_async_copy(samp, zs_h.at[:, pl.ds(0, CH)], sem_out).wait()
        pltpu.make_async_copy(zall, za_h.at[:, pl.ds(0, CH)], sem_out).wait()

    issue_in(0, zin0, win0, sem_in0)

    def pair_body(j, _):
        for b in (0, 1):
            ci = 2 * j + b
            zin, win, samp, zall, sem_in, sem_out = bufs[b]
            nzin, nwin, _ns, _nz, nsem_in, _nso = bufs[1 - b]
            wait_in(zin, win, sem_in)

            @pl.when(ci + 1 < NCHUNK)
            def _prefetch():
                issue_in(ci + 1, nzin, nwin, nsem_in)

            @pl.when(ci >= 2)
            def _drain():
                wait_out(samp, zall, sem_out)

            for r in range(CH):
                ray_body(r, zin, win, samp, zall)
            issue_out(ci, samp, zall, sem_out)
        return _

    lax.fori_loop(0, NCHUNK // 2, pair_body, 0)
    wait_out(samp0, zall0, sem_out0)
    wait_out(samp1, zall1, sem_out1)


@functools.lru_cache(maxsize=1)
def _make_sc_call():
    mesh = plsc.VectorSubcoreMesh(
        core_axis_name="c", subcore_axis_name="s",
        num_cores=2, num_subcores=16)
    return pl.kernel(
        _body,
        out_type=[
            jax.ShapeDtypeStruct((NALL, RAYS), _f32),
            jax.ShapeDtypeStruct((NIMP, RAYS), _f32),
        ],
        mesh=mesh,
        compiler_params=pltpu.CompilerParams(
            needs_layout_passes=False, use_tc_tiling_on_sc=False),
        scratch_types=[
            pltpu.VMEM((CH * 64 + 16,), _f32),
            pltpu.VMEM((CH * 64,), _f32),
            pltpu.VMEM((NIMP, CH), _f32),
            pltpu.VMEM((NALL, CH), _f32),
            pltpu.VMEM((CH * 64 + 16,), _f32),
            pltpu.VMEM((CH * 64,), _f32),
            pltpu.VMEM((NIMP, CH), _f32),
            pltpu.VMEM((NALL, CH), _f32),
            pltpu.VMEM((64,), _f32),
            pltpu.VMEM((64,), _f32),
            pltpu.VMEM((144,), _i32),
            pltpu.VMEM((80,), _i32),
            pltpu.SemaphoreType.DMA,
            pltpu.SemaphoreType.DMA,
            pltpu.SemaphoreType.DMA,
            pltpu.SemaphoreType.DMA,
        ],
    )


def _pts_body(o_ref, d_ref, z_ref, pts_ref):
    z = z_ref[...]
    for c in range(3):
        o = o_ref[c, :][None, :]
        d = d_ref[c, :][None, :]
        pts_ref[c, :, :] = o + d * z


_PB = 2048


def _tc_pts(ro_t, rd_t, za_t):
    return pl.pallas_call(
        _pts_body,
        out_shape=jax.ShapeDtypeStruct((3, NALL, RAYS), _f32),
        grid=(RAYS // _PB,),
        in_specs=[
            pl.BlockSpec((3, _PB), lambda i: (0, i)),
            pl.BlockSpec((3, _PB), lambda i: (0, i)),
            pl.BlockSpec((NALL, _PB), lambda i: (0, i)),
        ],
        out_specs=pl.BlockSpec((3, NALL, _PB), lambda i: (0, 0, i)),
    )(ro_t, rd_t, za_t)


@jax.jit
def kernel(rays_o, rays_d, z_vals, weights):
    za_t, zs_t = _make_sc_call()(z_vals.reshape(-1), weights.reshape(-1))
    pts_t = _tc_pts(rays_o.T, rays_d.T, za_t)
    return (jnp.transpose(pts_t, (2, 1, 0)), za_t.T, zs_t.T)
